# Initial kernel scaffold; baseline (speedup 1.0000x reference)
import jax
import jax.numpy as jnp
from jax.experimental import pallas as pl


def kernel(x, adj_intra, adj_masked, adj, y, labeled_nodes, params):
    raise NotImplementedError("write your pallas kernel here")


# speedup vs baseline: 11.3511x; 11.3511x over previous
import jax
import jax.numpy as jnp
from jax import lax
from jax.experimental import pallas as pl
from jax.experimental.pallas import tpu as pltpu
from jax.experimental.pallas import tpu_sc as plsc

ROWB = 256
LANES = 128
NTILE = 16
NCORE = 2


def _make_prep(npad, nchunk):
    npt = npad // NTILE
    cpt = nchunk // NTILE
    kpt = npt // LANES

    def body(dsts, permr, xpad, deg_out, xcor_out,
             idx_v, ones_v, zbuf, gbuf, perm_v, hist0, hist1, hist2):
        c = lax.axis_index("c")
        s = lax.axis_index("s")
        hists = (hist0, hist1, hist2)

        @pl.when(c == 0)
        def _hist():
            @pl.loop(0, LANES // 16)
            def _(i):
                ones_v[pl.ds(i * 16, 16)] = jnp.full((16,), 1.0, jnp.float32)

            @pl.loop(0, npt // 16)
            def _(i):
                zbuf[pl.ds(i * 16, 16)] = jnp.zeros((16,), jnp.float32)

            for a in range(3):
                pltpu.sync_copy(zbuf, hists[a].at[pl.ds(s * npt, npt)])
            plsc.subcore_barrier()
            for a in range(3):
                pltpu.sync_copy(dsts.at[a].at[pl.ds(s * cpt, cpt)], idx_v)

                @pl.loop(0, cpt)
                def _(j):
                    pltpu.sync_copy(ones_v, hists[a].at[idx_v.at[j]],
                                    add=True)
            plsc.subcore_barrier()
            for a in range(3):
                pltpu.sync_copy(hists[a].at[pl.ds(s * npt, npt)],
                                deg_out.at[pl.ds(a * npad + s * npt, npt)])

        @pl.when(c == 1)
        def _xcor():
            pltpu.sync_copy(permr.at[pl.ds(s * npt, npt)], perm_v)
            for k in range(kpt):
                pltpu.sync_copy(xpad.at[perm_v.at[pl.ds(k * LANES, LANES)]],
                                gbuf)
                pltpu.sync_copy(
                    gbuf, xcor_out.at[pl.ds(s * npt + k * LANES, LANES)])

    return pl.kernel(
        body,
        out_type=(
            jax.ShapeDtypeStruct((3 * npad,), jnp.float32),
            jax.ShapeDtypeStruct((npad, 256), jnp.float32),
        ),
        mesh=plsc.VectorSubcoreMesh(core_axis_name="c", subcore_axis_name="s"),
        scratch_types=[
            pltpu.VMEM((cpt, LANES), jnp.int32),
            pltpu.VMEM((LANES,), jnp.float32),
            pltpu.VMEM((npt,), jnp.float32),
            pltpu.VMEM((LANES, 256), jnp.float32),
            pltpu.VMEM((npt,), jnp.int32),
            pltpu.VMEM_SHARED((npad,), jnp.float32),
            pltpu.VMEM_SHARED((npad,), jnp.float32),
            pltpu.VMEM_SHARED((npad,), jnp.float32),
        ],
    )


def _make_spmm(npad, nchunk):
    npt = npad // NTILE
    cpt = nchunk // NTILE
    gsz = 16
    ngrp = cpt // gsz

    def body(h, srcs, dsts, out, src_v, dst_v, buf0, buf1, sem0, sem1, acc_sp):
        c = lax.axis_index("c")
        s = lax.axis_index("s")

        for a in range(3):
            for blk in range(2):
                b = 2 * c + blk
                hb = h.at[a, b]
                ob = out.at[a, b]
                pltpu.sync_copy(hb.at[pl.ds(s * npt, npt)],
                                acc_sp.at[pl.ds(s * npt, npt)])
                plsc.subcore_barrier()

                @pl.loop(0, ngrp)
                def _(g):
                    base = s * cpt + g * gsz
                    pltpu.sync_copy(srcs.at[a].at[pl.ds(base, gsz)], src_v)
                    pltpu.sync_copy(dsts.at[a].at[pl.ds(base, gsz)], dst_v)
                    pltpu.async_copy(hb.at[src_v.at[0]], buf0, sem0)

                    @pl.loop(0, gsz, step=2)
                    def _(j2):
                        pltpu.async_copy(hb.at[src_v.at[j2 + 1]], buf1, sem1)
                        pltpu.make_async_copy(
                            hb.at[src_v.at[0]], buf0, sem0).wait()
                        pltpu.sync_copy(buf0, acc_sp.at[dst_v.at[j2]],
                                        add=True)
                        pltpu.async_copy(
                            hb.at[src_v.at[jnp.minimum(j2 + 2, gsz - 1)]],
                            buf0, sem0)
                        pltpu.make_async_copy(
                            hb.at[src_v.at[0]], buf1, sem1).wait()
                        pltpu.sync_copy(buf1, acc_sp.at[dst_v.at[j2 + 1]],
                                        add=True)

                    pltpu.make_async_copy(hb.at[src_v.at[0]], buf0, sem0).wait()

                plsc.subcore_barrier()
                pltpu.sync_copy(acc_sp.at[pl.ds(s * npt, npt)],
                                ob.at[pl.ds(s * npt, npt)])

    return pl.kernel(
        body,
        out_type=jax.ShapeDtypeStruct((3, 4, npad, LANES), jnp.float32),
        mesh=plsc.VectorSubcoreMesh(core_axis_name="c", subcore_axis_name="s"),
        scratch_types=[
            pltpu.VMEM((gsz, LANES), jnp.int32),
            pltpu.VMEM((gsz, LANES), jnp.int32),
            pltpu.VMEM((LANES, LANES), jnp.float32),
            pltpu.VMEM((LANES, LANES), jnp.float32),
            pltpu.SemaphoreType.DMA,
            pltpu.SemaphoreType.DMA,
            pltpu.VMEM_SHARED((npad, LANES), jnp.float32),
        ],
    )


def _stage2_body(xp_ref, xc_ref, w1_ref, deg_ref, out_ref):
    d = lax.rsqrt(deg_ref[0] + 1.0)
    w = w1_ref[0]
    hcl = jnp.dot(xp_ref[...], w, preferred_element_type=jnp.float32) * d
    hco = jnp.dot(xc_ref[...], w, preferred_element_type=jnp.float32) * d
    out_ref[0, 0] = hcl[:, :LANES]
    out_ref[0, 1] = hcl[:, LANES:]
    out_ref[0, 2] = hco[:, :LANES]
    out_ref[0, 3] = hco[:, LANES:]


def _ln_leaky(hv, g, be, alpha):
    mu = jnp.mean(hv, axis=1, keepdims=True)
    xc = hv - mu
    var = jnp.mean(xc * xc, axis=1, keepdims=True)
    hn = xc * lax.rsqrt(var + 1e-5) * g + be
    return jnp.where(hn >= 0, hn, alpha * hn)


def _stage4_body(acc_ref, deg_ref, w2_ref, b1_ref, g1_ref, be1_ref, a1_ref,
                 out_ref):
    d = lax.rsqrt(deg_ref[0] + 1.0)
    w2 = w2_ref[0]
    for half in range(2):
        hv = jnp.concatenate(
            [acc_ref[0, 2 * half], acc_ref[0, 2 * half + 1]], axis=1)
        hv = hv * d + b1_ref[0, 0]
        hl = _ln_leaky(hv, g1_ref[0, 0], be1_ref[0, 0], a1_ref[0, 0, 0])
        h2 = jnp.dot(hl, w2, preferred_element_type=jnp.float32) * d
        out_ref[0, 2 * half] = h2[:, :LANES]
        out_ref[0, 2 * half + 1] = h2[:, LANES:]


def _stage6_body(acc_ref, deg_ref, b2_ref, g2_ref, be2_ref, a2_ref,
                 wm_ref, bm_ref, wpt_ref, bp_ref, out_ref):
    zs = []
    for a in range(3):
        d = lax.rsqrt(deg_ref[a] + 1.0)
        for half in range(2):
            hv = jnp.concatenate(
                [acc_ref[a, 2 * half], acc_ref[a, 2 * half + 1]], axis=1)
            hv = hv * d + b2_ref[a]
            zs.append(_ln_leaky(hv, g2_ref[a], be2_ref[a], a2_ref[a, 0]))
    z_i, z_is, z_m, z_ms, z_f, z_fs = zs
    cat = jnp.concatenate([z_i, z_m, z_f], axis=1)
    zmix = jnp.dot(cat, wm_ref[...], preferred_element_type=jnp.float32)
    zmix = jnp.maximum(zmix + bm_ref[0] + (z_i + z_m + z_f), 0.0)
    wpl = jnp.sum(wpt_ref[...], axis=0, keepdims=True)
    bps = jnp.sum(bp_ref[0])
    cols = [jnp.sum((zmix + t) * wpl, axis=1, keepdims=True) + bps
            for t in (z_i, z_m, z_f, z_is, z_ms, z_fs)]
    out_ref[...] = jnp.concatenate(cols, axis=1)


def _stage2(xpad, xcor, w1s, deg3, npad):
    gi = npad // ROWB
    return pl.pallas_call(
        _stage2_body,
        grid=(3, gi),
        in_specs=[
            pl.BlockSpec((ROWB, 256), lambda a, i: (i, 0)),
            pl.BlockSpec((ROWB, 256), lambda a, i: (i, 0)),
            pl.BlockSpec((1, 256, 256), lambda a, i: (a, 0, 0)),
            pl.BlockSpec((1, ROWB, 1), lambda a, i: (a, i, 0)),
        ],
        out_specs=pl.BlockSpec((1, 4, ROWB, LANES), lambda a, i: (a, 0, i, 0)),
        out_shape=jax.ShapeDtypeStruct((3, 4, npad, LANES), jnp.float32),
    )(xpad, xcor, w1s, deg3)


def _stage4(acc1, deg3, w2s, b1s, g1s, be1s, a1s, npad):
    gi = npad // ROWB
    vec = pl.BlockSpec((1, 1, 256), lambda a, i: (a, 0, 0))
    return pl.pallas_call(
        _stage4_body,
        grid=(3, gi),
        in_specs=[
            pl.BlockSpec((1, 4, ROWB, LANES), lambda a, i: (a, 0, i, 0)),
            pl.BlockSpec((1, ROWB, 1), lambda a, i: (a, i, 0)),
            pl.BlockSpec((1, 256, 256), lambda a, i: (a, 0, 0)),
            vec, vec, vec,
            pl.BlockSpec((1, 1, 1), lambda a, i: (a, 0, 0)),
        ],
        out_specs=pl.BlockSpec((1, 4, ROWB, LANES), lambda a, i: (a, 0, i, 0)),
        out_shape=jax.ShapeDtypeStruct((3, 4, npad, LANES), jnp.float32),
    )(acc1, deg3, w2s, b1s.reshape(3, 1, 256), g1s.reshape(3, 1, 256),
      be1s.reshape(3, 1, 256), a1s.reshape(3, 1, 1))


def _stage6(acc2, deg3, b2s, g2s, be2s, a2s, wm, bm, wpt, bp, npad):
    gi = npad // ROWB
    vec3 = pl.BlockSpec((3, 256), lambda i: (0, 0))
    return pl.pallas_call(
        _stage6_body,
        grid=(gi,),
        in_specs=[
            pl.BlockSpec((3, 4, ROWB, LANES), lambda i: (0, 0, i, 0)),
            pl.BlockSpec((3, ROWB, 1), lambda i: (0, i, 0)),
            vec3, vec3, vec3,
            pl.BlockSpec((3, 1), lambda i: (0, 0)),
            pl.BlockSpec((768, 256), lambda i: (0, 0)),
            pl.BlockSpec((1, 256), lambda i: (0, 0)),
            pl.BlockSpec((256, 256), lambda i: (0, 0)),
            pl.BlockSpec((1, 256), lambda i: (0, 0)),
        ],
        out_specs=pl.BlockSpec((ROWB, 6), lambda i: (i, 0)),
        out_shape=jax.ShapeDtypeStruct((npad, 6), jnp.float32),
    )(acc2, deg3, b2s, g2s, be2s, a2s, wm, bm, wpt, bp)


def kernel(x, adj_intra, adj_masked, adj, y, labeled_nodes, params):
    n, dfeat = x.shape
    e = adj_intra.shape[1]
    npad = -(-n // 2048) * 2048
    epad = -(-e // 4096) * 4096
    nchunk = epad // LANES

    perm = jax.random.permutation(jax.random.key(42), n).astype(jnp.int32)
    permr = jnp.concatenate(
        [perm, jnp.arange(npad - n, dtype=jnp.int32)])
    xpad = jnp.pad(x, ((0, npad - n), (0, 0)))

    pad_idx = n + jnp.arange(epad - e, dtype=jnp.int32) % (npad - n)

    def prep_edges(edges):
        src = jnp.concatenate([edges[0].astype(jnp.int32), pad_idx])
        dst = jnp.concatenate([edges[1].astype(jnp.int32), pad_idx])
        return src.reshape(nchunk, LANES), dst.reshape(nchunk, LANES)

    s_i, d_i = prep_edges(adj_intra)
    s_m, d_m = prep_edges(adj_masked)
    s_f, d_f = prep_edges(adj)
    srcs = jnp.stack([s_i, s_m, s_f])
    dsts = jnp.stack([d_i, d_m, d_f])

    deg, xcor = _make_prep(npad, nchunk)(dsts, permr, xpad)
    deg3 = deg.reshape(3, npad, 1)

    pe = [params["intra"], params["masked"], params["full"]]
    w1s = jnp.stack([p["W1"] for p in pe])
    w2s = jnp.stack([p["W2"] for p in pe])
    b1s = jnp.stack([p["b1"] for p in pe])
    g1s = jnp.stack([p["g1"] for p in pe])
    be1s = jnp.stack([p["be1"] for p in pe])
    a1s = jnp.stack([p["a1"] for p in pe]).reshape(3, 1)
    b2s = jnp.stack([p["b2"] for p in pe])
    g2s = jnp.stack([p["g2"] for p in pe])
    be2s = jnp.stack([p["be2"] for p in pe])
    a2s = jnp.stack([p["a2"] for p in pe]).reshape(3, 1)

    spmm = _make_spmm(npad, nchunk)
    h1 = _stage2(xpad, xcor, w1s, deg3, npad)
    acc1 = spmm(h1, srcs, dsts)
    h2 = _stage4(acc1, deg3, w2s, b1s, g1s, be1s, a1s, npad)
    acc2 = spmm(h2, srcs, dsts)
    out6 = _stage6(acc2, deg3, b2s, g2s, be2s, a2s,
                   params["Wm"], params["bm"].reshape(1, 256),
                   params["Wp"].T, params["bp"].reshape(1, 256), npad)
    return out6[:n].T.reshape(-1)


# speedup vs baseline: 12.3192x; 1.0853x over previous
import jax
import jax.numpy as jnp
from jax import lax
from jax.experimental import pallas as pl
from jax.experimental.pallas import tpu as pltpu
from jax.experimental.pallas import tpu_sc as plsc

ROWB = 256
LANES = 128
NTILE = 16
NCORE = 2


def _make_prep(npad, nchunk):
    npt = npad // NTILE
    cpt = nchunk // NTILE
    kpt = npt // LANES

    def body(d0, d1, d2, permr, xpad, deg_out, xcor_out,
             idx_v, ones_v, zbuf, gbuf, perm_v, hist0, hist1, hist2):
        c = lax.axis_index("c")
        s = lax.axis_index("s")
        dsts = (d0, d1, d2)
        hists = (hist0, hist1, hist2)

        @pl.when(c == 0)
        def _hist():
            @pl.loop(0, LANES // 16)
            def _(i):
                ones_v[pl.ds(i * 16, 16)] = jnp.full((16,), 1.0, jnp.float32)

            @pl.loop(0, npt // 16)
            def _(i):
                zbuf[pl.ds(i * 16, 16)] = jnp.zeros((16,), jnp.float32)

            for a in range(3):
                pltpu.sync_copy(zbuf, hists[a].at[pl.ds(s * npt, npt)])
            plsc.subcore_barrier()
            for a in range(3):
                pltpu.sync_copy(dsts[a].at[pl.ds(s * cpt, cpt)], idx_v)

                @pl.loop(0, cpt)
                def _(j):
                    pltpu.sync_copy(ones_v, hists[a].at[idx_v.at[j]],
                                    add=True)
            plsc.subcore_barrier()
            for a in range(3):
                pltpu.sync_copy(hists[a].at[pl.ds(s * npt, npt)],
                                deg_out.at[pl.ds(a * npad + s * npt, npt)])

        @pl.when(c == 1)
        def _xcor():
            pltpu.sync_copy(permr.at[pl.ds(s * npt, npt)], perm_v)
            for k in range(kpt):
                pltpu.sync_copy(xpad.at[perm_v.at[pl.ds(k * LANES, LANES)]],
                                gbuf)
                pltpu.sync_copy(
                    gbuf, xcor_out.at[pl.ds(s * npt + k * LANES, LANES)])

    return pl.kernel(
        body,
        out_type=(
            jax.ShapeDtypeStruct((3 * npad,), jnp.float32),
            jax.ShapeDtypeStruct((npad, 256), jnp.float32),
        ),
        mesh=plsc.VectorSubcoreMesh(core_axis_name="c", subcore_axis_name="s"),
        scratch_types=[
            pltpu.VMEM((cpt, LANES), jnp.int32),
            pltpu.VMEM((LANES,), jnp.float32),
            pltpu.VMEM((npt,), jnp.float32),
            pltpu.VMEM((LANES, 256), jnp.float32),
            pltpu.VMEM((npt,), jnp.int32),
            pltpu.VMEM_SHARED((npad,), jnp.float32),
            pltpu.VMEM_SHARED((npad,), jnp.float32),
            pltpu.VMEM_SHARED((npad,), jnp.float32),
        ],
    )


def _make_spmm(npad, nchunk):
    npt = npad // NTILE
    cpt = nchunk // NTILE
    gsz = 16
    ngrp = cpt // gsz

    def body(h, srcs, dsts, out, src_v, dst_v, buf0, buf1, sem0, sem1, acc_sp):
        c = lax.axis_index("c")
        s = lax.axis_index("s")

        for blk in range(2):
            b = 2 * c + blk
            hb = h.at[b]
            ob = out.at[b]
            pltpu.sync_copy(hb.at[pl.ds(s * npt, npt)],
                            acc_sp.at[pl.ds(s * npt, npt)])
            plsc.subcore_barrier()

            @pl.loop(0, ngrp)
            def _(g):
                base = s * cpt + g * gsz
                pltpu.sync_copy(srcs.at[pl.ds(base, gsz)], src_v)
                pltpu.sync_copy(dsts.at[pl.ds(base, gsz)], dst_v)
                pltpu.async_copy(hb.at[src_v.at[0]], buf0, sem0)

                @pl.loop(0, gsz, step=2)
                def _(j2):
                    pltpu.async_copy(hb.at[src_v.at[j2 + 1]], buf1, sem1)
                    pltpu.make_async_copy(
                        hb.at[src_v.at[0]], buf0, sem0).wait()
                    pltpu.sync_copy(buf0, acc_sp.at[dst_v.at[j2]],
                                    add=True)
                    pltpu.async_copy(
                        hb.at[src_v.at[jnp.minimum(j2 + 2, gsz - 1)]],
                        buf0, sem0)
                    pltpu.make_async_copy(
                        hb.at[src_v.at[0]], buf1, sem1).wait()
                    pltpu.sync_copy(buf1, acc_sp.at[dst_v.at[j2 + 1]],
                                    add=True)

                pltpu.make_async_copy(hb.at[src_v.at[0]], buf0, sem0).wait()

            plsc.subcore_barrier()
            pltpu.sync_copy(acc_sp.at[pl.ds(s * npt, npt)],
                            ob.at[pl.ds(s * npt, npt)])

    return pl.kernel(
        body,
        out_type=jax.ShapeDtypeStruct((4, npad, LANES), jnp.float32),
        mesh=plsc.VectorSubcoreMesh(core_axis_name="c", subcore_axis_name="s"),
        scratch_types=[
            pltpu.VMEM((16, LANES), jnp.int32),
            pltpu.VMEM((16, LANES), jnp.int32),
            pltpu.VMEM((LANES, LANES), jnp.float32),
            pltpu.VMEM((LANES, LANES), jnp.float32),
            pltpu.SemaphoreType.DMA,
            pltpu.SemaphoreType.DMA,
            pltpu.VMEM_SHARED((npad, LANES), jnp.float32),
        ],
    )


def _stage2_body(xp_ref, xc_ref, w1_ref, deg_ref, out_ref):
    d = lax.rsqrt(deg_ref[...] + 1.0)
    w = w1_ref[...]
    hcl = jnp.dot(xp_ref[...], w, preferred_element_type=jnp.float32) * d
    hco = jnp.dot(xc_ref[...], w, preferred_element_type=jnp.float32) * d
    out_ref[0] = hcl[:, :LANES]
    out_ref[1] = hcl[:, LANES:]
    out_ref[2] = hco[:, :LANES]
    out_ref[3] = hco[:, LANES:]


def _ln_leaky(hv, g, be, alpha):
    mu = jnp.mean(hv, axis=1, keepdims=True)
    xc = hv - mu
    var = jnp.mean(xc * xc, axis=1, keepdims=True)
    hn = xc * lax.rsqrt(var + 1e-5) * g + be
    return jnp.where(hn >= 0, hn, alpha * hn)


def _stage4_body(acc_ref, deg_ref, w2_ref, b1_ref, g1_ref, be1_ref, a1_ref,
                 out_ref):
    d = lax.rsqrt(deg_ref[...] + 1.0)
    w2 = w2_ref[...]
    for half in range(2):
        hv = jnp.concatenate(
            [acc_ref[2 * half], acc_ref[2 * half + 1]], axis=1)
        hv = hv * d + b1_ref[0]
        hl = _ln_leaky(hv, g1_ref[0], be1_ref[0], a1_ref[0, 0])
        h2 = jnp.dot(hl, w2, preferred_element_type=jnp.float32) * d
        out_ref[2 * half] = h2[:, :LANES]
        out_ref[2 * half + 1] = h2[:, LANES:]


def _stage6_body(acc_i, acc_m, acc_f, deg_ref, b2_ref, g2_ref, be2_ref,
                 a2_ref, wm_ref, bm_ref, wpt_ref, bp_ref, out_ref):
    accs = (acc_i, acc_m, acc_f)
    zs = []
    for a in range(3):
        d = lax.rsqrt(deg_ref[a] + 1.0)
        for half in range(2):
            hv = jnp.concatenate(
                [accs[a][2 * half], accs[a][2 * half + 1]], axis=1)
            hv = hv * d + b2_ref[a]
            zs.append(_ln_leaky(hv, g2_ref[a], be2_ref[a], a2_ref[a, 0]))
    z_i, z_is, z_m, z_ms, z_f, z_fs = zs
    cat = jnp.concatenate([z_i, z_m, z_f], axis=1)
    zmix = jnp.dot(cat, wm_ref[...], preferred_element_type=jnp.float32)
    zmix = jnp.maximum(zmix + bm_ref[0] + (z_i + z_m + z_f), 0.0)
    wpl = jnp.sum(wpt_ref[...], axis=0, keepdims=True)
    bps = jnp.sum(bp_ref[0])
    cols = [jnp.sum((zmix + t) * wpl, axis=1, keepdims=True) + bps
            for t in (z_i, z_m, z_f, z_is, z_ms, z_fs)]
    out_ref[...] = jnp.concatenate(cols, axis=1)


def _stage2(xpad, xcor, w1, deg_a, npad):
    gi = npad // ROWB
    return pl.pallas_call(
        _stage2_body,
        grid=(gi,),
        in_specs=[
            pl.BlockSpec((ROWB, 256), lambda i: (i, 0)),
            pl.BlockSpec((ROWB, 256), lambda i: (i, 0)),
            pl.BlockSpec((256, 256), lambda i: (0, 0)),
            pl.BlockSpec((ROWB, 1), lambda i: (i, 0)),
        ],
        out_specs=pl.BlockSpec((4, ROWB, LANES), lambda i: (0, i, 0)),
        out_shape=jax.ShapeDtypeStruct((4, npad, LANES), jnp.float32),
    )(xpad, xcor, w1, deg_a)


def _stage4(acc1, deg_a, p, npad):
    gi = npad // ROWB
    full = lambda *shape: pl.BlockSpec(shape, lambda i: (0,) * len(shape))
    return pl.pallas_call(
        _stage4_body,
        grid=(gi,),
        in_specs=[
            pl.BlockSpec((4, ROWB, LANES), lambda i: (0, i, 0)),
            pl.BlockSpec((ROWB, 1), lambda i: (i, 0)),
            full(256, 256),
            full(1, 256), full(1, 256), full(1, 256),
            full(1, 1),
        ],
        out_specs=pl.BlockSpec((4, ROWB, LANES), lambda i: (0, i, 0)),
        out_shape=jax.ShapeDtypeStruct((4, npad, LANES), jnp.float32),
    )(acc1, deg_a, p["W2"], p["b1"].reshape(1, 256), p["g1"].reshape(1, 256),
      p["be1"].reshape(1, 256), p["a1"].reshape(1, 1))


def _stage6(accs, deg3, b2s, g2s, be2s, a2s, wm, bm, wpt, bp, npad):
    gi = npad // ROWB
    full = lambda *shape: pl.BlockSpec(shape, lambda i: (0,) * len(shape))
    acc_spec = pl.BlockSpec((4, ROWB, LANES), lambda i: (0, i, 0))
    return pl.pallas_call(
        _stage6_body,
        grid=(gi,),
        in_specs=[
            acc_spec, acc_spec, acc_spec,
            pl.BlockSpec((3, ROWB, 1), lambda i: (0, i, 0)),
            full(3, 256), full(3, 256), full(3, 256),
            full(3, 1),
            full(768, 256),
            full(1, 256),
            full(256, 256),
            full(1, 256),
        ],
        out_specs=pl.BlockSpec((ROWB, 6), lambda i: (i, 0)),
        out_shape=jax.ShapeDtypeStruct((npad, 6), jnp.float32),
    )(*accs, deg3, b2s, g2s, be2s, a2s, wm, bm, wpt, bp)


def kernel(x, adj_intra, adj_masked, adj, y, labeled_nodes, params):
    n, dfeat = x.shape
    e = adj_intra.shape[1]
    npad = -(-n // 2048) * 2048
    epad = -(-e // 4096) * 4096
    nchunk = epad // LANES

    perm = jax.random.permutation(jax.random.key(42), n).astype(jnp.int32)
    permr = jnp.concatenate(
        [perm, jnp.arange(npad - n, dtype=jnp.int32)])
    xpad = jnp.pad(x, ((0, npad - n), (0, 0)))

    pad_idx = n + jnp.arange(epad - e, dtype=jnp.int32) % (npad - n)

    def prep_edges(edges):
        src = jnp.concatenate([edges[0].astype(jnp.int32), pad_idx])
        dst = jnp.concatenate([edges[1].astype(jnp.int32), pad_idx])
        return src.reshape(nchunk, LANES), dst.reshape(nchunk, LANES)

    srcs = []
    dsts = []
    for edges in (adj_intra, adj_masked, adj):
        sa, da = prep_edges(edges)
        srcs.append(sa)
        dsts.append(da)

    deg, xcor = _make_prep(npad, nchunk)(dsts[0], dsts[1], dsts[2],
                                         permr, xpad)
    deg3 = deg.reshape(3, npad, 1)

    pe = [params["intra"], params["masked"], params["full"]]
    spmm = _make_spmm(npad, nchunk)

    h1 = [_stage2(xpad, xcor, pe[a]["W1"], deg3[a], npad) for a in range(3)]
    acc1 = [spmm(h1[a], srcs[a], dsts[a]) for a in range(3)]
    h2 = [_stage4(acc1[a], deg3[a], pe[a], npad) for a in range(3)]
    acc2 = [spmm(h2[a], srcs[a], dsts[a]) for a in range(3)]

    b2s = jnp.stack([p["b2"] for p in pe])
    g2s = jnp.stack([p["g2"] for p in pe])
    be2s = jnp.stack([p["be2"] for p in pe])
    a2s = jnp.stack([p["a2"] for p in pe]).reshape(3, 1)
    out6 = _stage6(acc2, deg3, b2s, g2s, be2s, a2s,
                   params["Wm"], params["bm"].reshape(1, 256),
                   params["Wp"].T, params["bp"].reshape(1, 256), npad)
    return out6[:n].T.reshape(-1)
